# Initial kernel scaffold; baseline (speedup 1.0000x reference)
import jax
import jax.numpy as jnp
from jax.experimental import pallas as pl


def kernel(w_qkv, w_o, hidden_states, sequence_mask):
    raise NotImplementedError("write your pallas kernel here")


# speedup vs baseline: 1.6087x; 1.6087x over previous
import functools

import jax
import jax.numpy as jnp
import numpy as np
from jax.experimental import pallas as pl
from jax.experimental.pallas import tpu as pltpu

NUM_HEADS = 16
NUM_KV_HEADS = 4
GROUP = NUM_HEADS // NUM_KV_HEADS


def _rotary_tables_np(seq_len, dim, theta=10000.0):
    inv_freq = 1.0 / (theta ** (np.arange(0, dim, 2, dtype=np.float64) / dim))
    ang = np.outer(np.arange(seq_len, dtype=np.float64), inv_freq)
    cos = np.cos(ang).astype(np.float32)
    sin = np.sin(ang).astype(np.float32)
    cos_cat = np.concatenate([cos, cos], axis=-1)
    sin_cat = np.concatenate([-sin, sin], axis=-1)
    return cos_cat, sin_cat


def _qkv_rot_kernel(x_ref, w_ref, cos_ref, sin_ref, o_ref, *, heads_per_tile,
                    scale):
    j = pl.program_id(1)
    acc = jnp.dot(x_ref[...], w_ref[...], preferred_element_type=jnp.float32)
    tm, tn = acc.shape
    nh = heads_per_tile
    d = tn // nh
    a3 = acc.reshape(tm, nh, d)
    cos = cos_ref[...].reshape(tm, 1, d)
    sin = sin_ref[...].reshape(tm, 1, d)
    swapped = jnp.concatenate([a3[:, :, d // 2:], a3[:, :, :d // 2]], axis=2)
    rot = a3 * cos + swapped * sin
    hidx = j * nh + jax.lax.broadcasted_iota(jnp.int32, (1, nh, 1), 1)
    coef = jnp.where(hidx < NUM_HEADS, jnp.float32(scale), jnp.float32(1.0))
    out3 = jnp.where(hidx < NUM_HEADS + NUM_KV_HEADS, rot * coef, a3)
    o_ref[...] = out3.reshape(tm, tn).astype(o_ref.dtype)


def _qkv_proj_rotary(x, w, cos_tab, sin_tab, *, seq, scale, tn=1024):
    M, K = x.shape
    _, N = w.shape
    tm = seq
    d = cos_tab.shape[-1]
    heads_per_tile = tn // d
    kern = functools.partial(_qkv_rot_kernel, heads_per_tile=heads_per_tile,
                             scale=scale)
    return pl.pallas_call(
        kern,
        out_shape=jax.ShapeDtypeStruct((M, N), jnp.bfloat16),
        grid=(M // tm, N // tn),
        in_specs=[
            pl.BlockSpec((tm, K), lambda i, j: (i, 0)),
            pl.BlockSpec((K, tn), lambda i, j: (0, j)),
            pl.BlockSpec((seq, d), lambda i, j: (0, 0)),
            pl.BlockSpec((seq, d), lambda i, j: (0, 0)),
        ],
        out_specs=pl.BlockSpec((tm, tn), lambda i, j: (i, j)),
        compiler_params=pltpu.CompilerParams(
            dimension_semantics=("parallel", "arbitrary"),
            vmem_limit_bytes=56 * 1024 * 1024),
    )(x, w, cos_tab, sin_tab)


def _attn_kernel(qi_tab, ki_tab, q_ref, k_ref, v_ref, o_ref,
                 m_scr, l_scr, acc_scr, *, ts, d):
    s_idx = pl.program_id(2)
    qi = qi_tab[s_idx]
    ki = ki_tab[s_idx]

    @pl.when(ki == 0)
    def _init():
        m_scr[...] = jnp.full(m_scr.shape, -jnp.inf, jnp.float32)
        l_scr[...] = jnp.zeros(l_scr.shape, jnp.float32)
        acc_scr[...] = jnp.zeros(acc_scr.shape, jnp.float32)

    k = k_ref[...]
    v = v_ref[...]
    on_diag = ki == qi
    row = jax.lax.broadcasted_iota(jnp.int32, (ts, ts), 0)
    col = jax.lax.broadcasted_iota(jnp.int32, (ts, ts), 1)
    keep = jnp.logical_or(jnp.logical_not(on_diag), col <= row)

    for u in range(GROUP):
        sl = slice(u * d, (u + 1) * d)
        q_u = q_ref[:, sl]
        s = jax.lax.dot_general(q_u, k, (((1,), (1,)), ((), ())),
                                preferred_element_type=jnp.float32)
        s = jnp.where(keep, s, -jnp.inf)
        m_prev = m_scr[u]
        m_new = jnp.maximum(m_prev, jnp.max(s, axis=-1, keepdims=True))
        alpha = jnp.exp(m_prev - m_new)
        p = jnp.exp(s - m_new)
        l_scr[u] = alpha * l_scr[u] + jnp.sum(p, axis=-1, keepdims=True)
        acc_scr[:, sl] = alpha * acc_scr[:, sl] + jnp.dot(
            p.astype(v.dtype), v, preferred_element_type=jnp.float32)
        m_scr[u] = m_new

    @pl.when(on_diag)
    def _fin():
        for u in range(GROUP):
            sl = slice(u * d, (u + 1) * d)
            inv = pl.reciprocal(l_scr[u], approx=True)
            o_ref[:, sl] = (acc_scr[:, sl] * inv).astype(o_ref.dtype)


def _flash_attention(qkv, *, batch, seq, head_dim, ts=256):
    d = head_dim
    n_blk = seq // ts
    pairs = [(qi, ki) for qi in range(n_blk) for ki in range(qi + 1)]
    qi_tab = jnp.asarray([p[0] for p in pairs], dtype=jnp.int32)
    ki_tab = jnp.asarray([p[1] for p in pairs], dtype=jnp.int32)
    n_steps = len(pairs)

    def q_map(b, g, s, qt, kt):
        return (b * n_blk + qt[s], g)

    def k_map(b, g, s, qt, kt):
        return (b * n_blk + kt[s], NUM_HEADS + g)

    def v_map(b, g, s, qt, kt):
        return (b * n_blk + kt[s], NUM_HEADS + NUM_KV_HEADS + g)

    def o_map(b, g, s, qt, kt):
        return (b * n_blk + qt[s], g)

    kern = functools.partial(_attn_kernel, ts=ts, d=d)
    return pl.pallas_call(
        kern,
        out_shape=jax.ShapeDtypeStruct((batch * seq, NUM_HEADS * d),
                                       jnp.bfloat16),
        grid_spec=pltpu.PrefetchScalarGridSpec(
            num_scalar_prefetch=2,
            grid=(batch, NUM_KV_HEADS, n_steps),
            in_specs=[
                pl.BlockSpec((ts, GROUP * d), q_map),
                pl.BlockSpec((ts, d), k_map),
                pl.BlockSpec((ts, d), v_map),
            ],
            out_specs=pl.BlockSpec((ts, GROUP * d), o_map),
            scratch_shapes=[
                pltpu.VMEM((GROUP, ts, 1), jnp.float32),
                pltpu.VMEM((GROUP, ts, 1), jnp.float32),
                pltpu.VMEM((ts, GROUP * d), jnp.float32),
            ]),
        compiler_params=pltpu.CompilerParams(
            dimension_semantics=("parallel", "parallel", "arbitrary"),
            vmem_limit_bytes=56 * 1024 * 1024),
    )(qi_tab, ki_tab, qkv, qkv, qkv)


def _mm_kernel(x_ref, w_ref, o_ref):
    o_ref[...] = jnp.dot(x_ref[...], w_ref[...],
                         preferred_element_type=jnp.float32).astype(o_ref.dtype)


def _matmul(x, w, *, tm=1024, tn=1024, out_dtype=jnp.float32):
    M, K = x.shape
    _, N = w.shape
    return pl.pallas_call(
        _mm_kernel,
        out_shape=jax.ShapeDtypeStruct((M, N), out_dtype),
        grid=(M // tm, N // tn),
        in_specs=[
            pl.BlockSpec((tm, K), lambda i, j: (i, 0)),
            pl.BlockSpec((K, tn), lambda i, j: (0, j)),
        ],
        out_specs=pl.BlockSpec((tm, tn), lambda i, j: (i, j)),
        compiler_params=pltpu.CompilerParams(
            dimension_semantics=("parallel", "arbitrary"),
            vmem_limit_bytes=56 * 1024 * 1024),
    )(x, w)


def kernel(w_qkv, w_o, hidden_states, sequence_mask):
    S, B, Hd = hidden_states.shape
    d = Hd // NUM_HEADS
    scale = 1.0 / (d ** 0.5)

    x = jnp.transpose(hidden_states, (1, 0, 2)).reshape(B * S, Hd)
    x = x.astype(jnp.bfloat16)
    wq = w_qkv.astype(jnp.bfloat16)
    wo = w_o.astype(jnp.bfloat16)

    cos_np, sin_np = _rotary_tables_np(S, d)
    cos_tab = jnp.asarray(cos_np)
    sin_tab = jnp.asarray(sin_np)

    qkv = _qkv_proj_rotary(x, wq, cos_tab, sin_tab, seq=S, scale=scale)
    attn = _flash_attention(qkv, batch=B, seq=S, head_dim=d)
    out = _matmul(attn, wo)
    out = jnp.transpose(out.reshape(B, S, Hd), (1, 0, 2))
    return {"hidden_states": out, "sequence_mask": sequence_mask}


# speedup vs baseline: 2.8258x; 1.7566x over previous
import functools

import jax
import jax.numpy as jnp
import numpy as np
from jax.experimental import pallas as pl
from jax.experimental.pallas import tpu as pltpu

NUM_HEADS = 16
NUM_KV_HEADS = 4
GROUP = NUM_HEADS // NUM_KV_HEADS


def _rotary_epilogue_tables_np(seq_len, dim, n_heads_tile, scale,
                               theta=10000.0):
    half = dim // 2
    inv_freq = 1.0 / (theta ** (np.arange(0, dim, 2, dtype=np.float64) / dim))
    ang = np.outer(np.arange(seq_len, dtype=np.float64), inv_freq)
    cos = np.cos(ang)
    sin = np.sin(ang)
    cos_cat = np.concatenate([cos, cos], axis=-1)
    sin_cat = np.concatenate([-sin, sin], axis=-1)
    sin_lo = np.concatenate([-sin, np.zeros_like(sin)], axis=-1)
    sin_hi = np.concatenate([np.zeros_like(sin), sin], axis=-1)
    del cos_cat, half

    def tile_q(t):
        return np.tile(t, (1, n_heads_tile)) * scale

    nk = n_heads_tile // 2
    ones_v = np.ones((seq_len, nk * dim))
    zeros_v = np.zeros((seq_len, nk * dim))

    cos_tab = np.concatenate(
        [tile_q(np.concatenate([cos, cos], -1)),
         np.concatenate([np.tile(np.concatenate([cos, cos], -1), (1, nk)),
                         ones_v], -1)], axis=-1)
    sin_lo_tab = np.concatenate(
        [tile_q(sin_lo),
         np.concatenate([np.tile(sin_lo, (1, nk)), zeros_v], -1)], axis=-1)
    sin_hi_tab = np.concatenate(
        [tile_q(sin_hi),
         np.concatenate([np.tile(sin_hi, (1, nk)), zeros_v], -1)], axis=-1)
    del sin_cat
    return (cos_tab.astype(np.float32), sin_lo_tab.astype(np.float32),
            sin_hi_tab.astype(np.float32))


def _qkv_rot_kernel(x_ref, w_ref, cos_ref, slo_ref, shi_ref, o_ref, *, half,
                    tn):
    j = pl.program_id(1)
    acc = jnp.dot(x_ref[...], w_ref[...], preferred_element_type=jnp.float32)
    sel = (j // 2) * tn
    cos = cos_ref[:, pl.ds(sel, tn)]
    slo = slo_ref[:, pl.ds(sel, tn)]
    shi = shi_ref[:, pl.ds(sel, tn)]
    r_lo = pltpu.roll(acc, shift=tn - half, axis=1)
    r_hi = pltpu.roll(acc, shift=half, axis=1)
    out = acc * cos + r_lo * slo + r_hi * shi
    o_ref[...] = out.astype(o_ref.dtype)


def _qkv_proj_rotary(x, w, tabs, *, seq, tn=1024):
    M, K = x.shape
    _, N = w.shape
    tm = seq
    d = 2 * tn
    kern = functools.partial(_qkv_rot_kernel, half=64, tn=tn)
    return pl.pallas_call(
        kern,
        out_shape=jax.ShapeDtypeStruct((M, N), jnp.bfloat16),
        grid=(M // tm, N // tn),
        in_specs=[
            pl.BlockSpec((tm, K), lambda i, j: (i, 0)),
            pl.BlockSpec((K, tn), lambda i, j: (0, j)),
            pl.BlockSpec((seq, d), lambda i, j: (0, 0)),
            pl.BlockSpec((seq, d), lambda i, j: (0, 0)),
            pl.BlockSpec((seq, d), lambda i, j: (0, 0)),
        ],
        out_specs=pl.BlockSpec((tm, tn), lambda i, j: (i, j)),
        compiler_params=pltpu.CompilerParams(
            dimension_semantics=("parallel", "arbitrary"),
            vmem_limit_bytes=56 * 1024 * 1024),
    )(x, w, *tabs)


def _attn_kernel(qi_tab, ki_tab, q_ref, k_ref, v_ref, o_ref,
                 m_scr, l_scr, acc_scr, *, ts, d):
    s_idx = pl.program_id(2)
    qi = qi_tab[s_idx]
    ki = ki_tab[s_idx]

    @pl.when(ki == 0)
    def _init():
        m_scr[...] = jnp.full(m_scr.shape, -jnp.inf, jnp.float32)
        l_scr[...] = jnp.zeros(l_scr.shape, jnp.float32)
        acc_scr[...] = jnp.zeros(acc_scr.shape, jnp.float32)

    k = k_ref[...]
    v = v_ref[...]
    on_diag = ki == qi
    row = jax.lax.broadcasted_iota(jnp.int32, (ts, ts), 0)
    col = jax.lax.broadcasted_iota(jnp.int32, (ts, ts), 1)
    keep = jnp.logical_or(jnp.logical_not(on_diag), col <= row)

    s_all = [
        jax.lax.dot_general(q_ref[:, u * d:(u + 1) * d], k,
                            (((1,), (1,)), ((), ())),
                            preferred_element_type=jnp.float32)
        for u in range(GROUP)
    ]
    for u in range(GROUP):
        sl = slice(u * d, (u + 1) * d)
        s = jnp.where(keep, s_all[u], -jnp.inf)
        m_prev = m_scr[u]
        m_new = jnp.maximum(m_prev, jnp.max(s, axis=-1, keepdims=True))
        alpha = jnp.exp(m_prev - m_new)
        p = jnp.exp(s - m_new)
        l_scr[u] = alpha * l_scr[u] + jnp.sum(p, axis=-1, keepdims=True)
        acc_scr[:, sl] = alpha * acc_scr[:, sl] + jnp.dot(
            p.astype(v.dtype), v, preferred_element_type=jnp.float32)
        m_scr[u] = m_new

    @pl.when(on_diag)
    def _fin():
        for u in range(GROUP):
            sl = slice(u * d, (u + 1) * d)
            inv = pl.reciprocal(l_scr[u], approx=True)
            o_ref[:, sl] = (acc_scr[:, sl] * inv).astype(o_ref.dtype)


def _flash_attention(qkv, *, batch, seq, head_dim, ts=512):
    d = head_dim
    n_blk = seq // ts
    pairs = [(qi, ki) for qi in range(n_blk) for ki in range(qi + 1)]
    qi_tab = jnp.asarray([p[0] for p in pairs], dtype=jnp.int32)
    ki_tab = jnp.asarray([p[1] for p in pairs], dtype=jnp.int32)
    n_steps = len(pairs)

    def q_map(b, g, s, qt, kt):
        return (b * n_blk + qt[s], g)

    def k_map(b, g, s, qt, kt):
        return (b * n_blk + kt[s], NUM_HEADS + g)

    def v_map(b, g, s, qt, kt):
        return (b * n_blk + kt[s], NUM_HEADS + NUM_KV_HEADS + g)

    def o_map(b, g, s, qt, kt):
        return (b * n_blk + qt[s], g)

    kern = functools.partial(_attn_kernel, ts=ts, d=d)
    return pl.pallas_call(
        kern,
        out_shape=jax.ShapeDtypeStruct((batch * seq, NUM_HEADS * d),
                                       jnp.bfloat16),
        grid_spec=pltpu.PrefetchScalarGridSpec(
            num_scalar_prefetch=2,
            grid=(batch, NUM_KV_HEADS, n_steps),
            in_specs=[
                pl.BlockSpec((ts, GROUP * d), q_map),
                pl.BlockSpec((ts, d), k_map),
                pl.BlockSpec((ts, d), v_map),
            ],
            out_specs=pl.BlockSpec((ts, GROUP * d), o_map),
            scratch_shapes=[
                pltpu.VMEM((GROUP, ts, 1), jnp.float32),
                pltpu.VMEM((GROUP, ts, 1), jnp.float32),
                pltpu.VMEM((ts, GROUP * d), jnp.float32),
            ]),
        compiler_params=pltpu.CompilerParams(
            dimension_semantics=("parallel", "parallel", "arbitrary"),
            vmem_limit_bytes=56 * 1024 * 1024),
    )(qi_tab, ki_tab, qkv, qkv, qkv)


def _mm_kernel(x_ref, w_ref, o_ref):
    o_ref[...] = jnp.dot(x_ref[...], w_ref[...],
                         preferred_element_type=jnp.float32).astype(o_ref.dtype)


def _matmul(x, w, *, tm=1024, tn=1024, out_dtype=jnp.float32):
    M, K = x.shape
    _, N = w.shape
    return pl.pallas_call(
        _mm_kernel,
        out_shape=jax.ShapeDtypeStruct((M, N), out_dtype),
        grid=(M // tm, N // tn),
        in_specs=[
            pl.BlockSpec((tm, K), lambda i, j: (i, 0)),
            pl.BlockSpec((K, tn), lambda i, j: (0, j)),
        ],
        out_specs=pl.BlockSpec((tm, tn), lambda i, j: (i, j)),
        compiler_params=pltpu.CompilerParams(
            dimension_semantics=("parallel", "arbitrary"),
            vmem_limit_bytes=56 * 1024 * 1024),
    )(x, w)


def kernel(w_qkv, w_o, hidden_states, sequence_mask):
    S, B, Hd = hidden_states.shape
    d = Hd // NUM_HEADS
    scale = 1.0 / (d ** 0.5)

    x = jnp.transpose(hidden_states, (1, 0, 2)).reshape(B * S, Hd)
    x = x.astype(jnp.bfloat16)
    wq = w_qkv.astype(jnp.bfloat16)
    wo = w_o.astype(jnp.bfloat16)

    tn = 1024
    tabs_np = _rotary_epilogue_tables_np(S, d, tn // d, scale)
    tabs = tuple(jnp.asarray(t, jnp.bfloat16) for t in tabs_np)

    qkv = _qkv_proj_rotary(x, wq, tabs, seq=S, tn=tn)
    attn = _flash_attention(qkv, batch=B, seq=S, head_dim=d)
    out = _matmul(attn, wo)
    out = jnp.transpose(out.reshape(B, S, Hd), (1, 0, 2))
    return {"hidden_states": out, "sequence_mask": sequence_mask}


# speedup vs baseline: 2.8399x; 1.0050x over previous
import functools

import jax
import jax.numpy as jnp
import numpy as np
from jax.experimental import pallas as pl
from jax.experimental.pallas import tpu as pltpu

NUM_HEADS = 16
NUM_KV_HEADS = 4
GROUP = NUM_HEADS // NUM_KV_HEADS


def _rotary_epilogue_tables_np(seq_len, dim, n_heads_tile, scale,
                               theta=10000.0):
    half = dim // 2
    inv_freq = 1.0 / (theta ** (np.arange(0, dim, 2, dtype=np.float64) / dim))
    ang = np.outer(np.arange(seq_len, dtype=np.float64), inv_freq)
    cos = np.cos(ang)
    sin = np.sin(ang)
    cos_cat = np.concatenate([cos, cos], axis=-1)
    sin_cat = np.concatenate([-sin, sin], axis=-1)
    sin_lo = np.concatenate([-sin, np.zeros_like(sin)], axis=-1)
    sin_hi = np.concatenate([np.zeros_like(sin), sin], axis=-1)
    del cos_cat, half

    def tile_q(t):
        return np.tile(t, (1, n_heads_tile)) * scale

    nk = n_heads_tile // 2
    ones_v = np.ones((seq_len, nk * dim))
    zeros_v = np.zeros((seq_len, nk * dim))

    cos_tab = np.concatenate(
        [tile_q(np.concatenate([cos, cos], -1)),
         np.concatenate([np.tile(np.concatenate([cos, cos], -1), (1, nk)),
                         ones_v], -1)], axis=-1)
    sin_lo_tab = np.concatenate(
        [tile_q(sin_lo),
         np.concatenate([np.tile(sin_lo, (1, nk)), zeros_v], -1)], axis=-1)
    sin_hi_tab = np.concatenate(
        [tile_q(sin_hi),
         np.concatenate([np.tile(sin_hi, (1, nk)), zeros_v], -1)], axis=-1)
    del sin_cat
    return (cos_tab.astype(np.float32), sin_lo_tab.astype(np.float32),
            sin_hi_tab.astype(np.float32))


def _qkv_rot_kernel(x_ref, w_ref, cos_ref, slo_ref, shi_ref, o_ref, xb_scr,
                    *, half, tn):
    j = pl.program_id(1)

    @pl.when(j == 0)
    def _cast():
        xb_scr[...] = x_ref[...].astype(xb_scr.dtype)

    acc = jnp.dot(xb_scr[...], w_ref[...], preferred_element_type=jnp.float32)
    sel = (j // 2) * tn
    cos = cos_ref[:, pl.ds(sel, tn)]
    slo = slo_ref[:, pl.ds(sel, tn)]
    shi = shi_ref[:, pl.ds(sel, tn)]
    r_lo = pltpu.roll(acc, shift=tn - half, axis=1)
    r_hi = pltpu.roll(acc, shift=half, axis=1)
    out = acc * cos + r_lo * slo + r_hi * shi
    o_ref[...] = out.astype(o_ref.dtype)


def _qkv_proj_rotary(x2d, w, tabs, *, seq, tn=1024):
    S, BHd = x2d.shape
    K, N = w.shape
    B = BHd // K
    tm = seq
    d = 2 * tn
    kern = functools.partial(_qkv_rot_kernel, half=64, tn=tn)
    return pl.pallas_call(
        kern,
        out_shape=jax.ShapeDtypeStruct((B * seq, N), jnp.bfloat16),
        grid=(B, N // tn),
        in_specs=[
            pl.BlockSpec((S, K), lambda i, j: (0, i)),
            pl.BlockSpec((K, tn), lambda i, j: (0, j)),
            pl.BlockSpec((seq, d), lambda i, j: (0, 0)),
            pl.BlockSpec((seq, d), lambda i, j: (0, 0)),
            pl.BlockSpec((seq, d), lambda i, j: (0, 0)),
        ],
        out_specs=pl.BlockSpec((tm, tn), lambda i, j: (i, j)),
        scratch_shapes=[pltpu.VMEM((S, K), jnp.bfloat16)],
        compiler_params=pltpu.CompilerParams(
            dimension_semantics=("parallel", "arbitrary"),
            vmem_limit_bytes=56 * 1024 * 1024),
    )(x2d, w, *tabs)


def _attn_kernel(qi_tab, ki_tab, q_ref, k_ref, v_ref, o_ref,
                 m_scr, l_scr, acc_scr, *, ts, d):
    s_idx = pl.program_id(2)
    qi = qi_tab[s_idx]
    ki = ki_tab[s_idx]

    @pl.when(ki == 0)
    def _init():
        m_scr[...] = jnp.full(m_scr.shape, -jnp.inf, jnp.float32)
        l_scr[...] = jnp.zeros(l_scr.shape, jnp.float32)
        acc_scr[...] = jnp.zeros(acc_scr.shape, jnp.float32)

    k = k_ref[...]
    v = v_ref[...]
    on_diag = ki == qi
    row = jax.lax.broadcasted_iota(jnp.int32, (ts, ts), 0)
    col = jax.lax.broadcasted_iota(jnp.int32, (ts, ts), 1)
    keep = jnp.logical_or(jnp.logical_not(on_diag), col <= row)

    s_all = [
        jax.lax.dot_general(q_ref[:, u * d:(u + 1) * d], k,
                            (((1,), (1,)), ((), ())),
                            preferred_element_type=jnp.float32)
        for u in range(GROUP)
    ]
    for u in range(GROUP):
        sl = slice(u * d, (u + 1) * d)
        s = jnp.where(keep, s_all[u], -jnp.inf)
        m_prev = m_scr[u]
        m_new = jnp.maximum(m_prev, jnp.max(s, axis=-1, keepdims=True))
        alpha = jnp.exp(m_prev - m_new)
        p = jnp.exp(s - m_new)
        l_scr[u] = alpha * l_scr[u] + jnp.sum(p, axis=-1, keepdims=True)
        acc_scr[:, sl] = alpha * acc_scr[:, sl] + jnp.dot(
            p.astype(v.dtype), v, preferred_element_type=jnp.float32)
        m_scr[u] = m_new

    @pl.when(on_diag)
    def _fin():
        for u in range(GROUP):
            sl = slice(u * d, (u + 1) * d)
            inv = pl.reciprocal(l_scr[u], approx=True)
            o_ref[:, sl] = (acc_scr[:, sl] * inv).astype(o_ref.dtype)


def _flash_attention(qkv, *, batch, seq, head_dim, ts=512):
    d = head_dim
    n_blk = seq // ts
    pairs = [(qi, ki) for qi in range(n_blk) for ki in range(qi + 1)]
    qi_tab = jnp.asarray([p[0] for p in pairs], dtype=jnp.int32)
    ki_tab = jnp.asarray([p[1] for p in pairs], dtype=jnp.int32)
    n_steps = len(pairs)

    def q_map(b, g, s, qt, kt):
        return (b * n_blk + qt[s], g)

    def k_map(b, g, s, qt, kt):
        return (b * n_blk + kt[s], NUM_HEADS + g)

    def v_map(b, g, s, qt, kt):
        return (b * n_blk + kt[s], NUM_HEADS + NUM_KV_HEADS + g)

    def o_map(b, g, s, qt, kt):
        return (b * n_blk + qt[s], g)

    kern = functools.partial(_attn_kernel, ts=ts, d=d)
    return pl.pallas_call(
        kern,
        out_shape=jax.ShapeDtypeStruct((batch * seq, NUM_HEADS * d),
                                       jnp.bfloat16),
        grid_spec=pltpu.PrefetchScalarGridSpec(
            num_scalar_prefetch=2,
            grid=(batch, NUM_KV_HEADS, n_steps),
            in_specs=[
                pl.BlockSpec((ts, GROUP * d), q_map),
                pl.BlockSpec((ts, d), k_map),
                pl.BlockSpec((ts, d), v_map),
            ],
            out_specs=pl.BlockSpec((ts, GROUP * d), o_map),
            scratch_shapes=[
                pltpu.VMEM((GROUP, ts, 1), jnp.float32),
                pltpu.VMEM((GROUP, ts, 1), jnp.float32),
                pltpu.VMEM((ts, GROUP * d), jnp.float32),
            ]),
        compiler_params=pltpu.CompilerParams(
            dimension_semantics=("parallel", "parallel", "arbitrary"),
            vmem_limit_bytes=56 * 1024 * 1024),
    )(qi_tab, ki_tab, qkv, qkv, qkv)


def _mm_kernel(x_ref, w_ref, o_ref):
    o_ref[...] = jnp.dot(x_ref[...], w_ref[...],
                         preferred_element_type=jnp.float32).astype(o_ref.dtype)


def _o_proj(x, w, *, seq, tn=1024, out_dtype=jnp.float32):
    M, K = x.shape
    _, N = w.shape
    B = M // seq
    nj = N // tn
    return pl.pallas_call(
        _mm_kernel,
        out_shape=jax.ShapeDtypeStruct((seq, B * N), out_dtype),
        grid=(B, nj),
        in_specs=[
            pl.BlockSpec((seq, K), lambda i, j: (i, 0)),
            pl.BlockSpec((K, tn), lambda i, j: (0, j)),
        ],
        out_specs=pl.BlockSpec((seq, tn), lambda i, j, _nj=nj: (0, i * _nj + j)),
        compiler_params=pltpu.CompilerParams(
            dimension_semantics=("parallel", "arbitrary"),
            vmem_limit_bytes=56 * 1024 * 1024),
    )(x, w)


def kernel(w_qkv, w_o, hidden_states, sequence_mask):
    S, B, Hd = hidden_states.shape
    d = Hd // NUM_HEADS
    scale = 1.0 / (d ** 0.5)

    x2d = hidden_states.reshape(S, B * Hd)
    wq = w_qkv.astype(jnp.bfloat16)
    wo = w_o.astype(jnp.bfloat16)

    tn = 1024
    tabs_np = _rotary_epilogue_tables_np(S, d, tn // d, scale)
    tabs = tuple(jnp.asarray(t, jnp.bfloat16) for t in tabs_np)

    qkv = _qkv_proj_rotary(x2d, wq, tabs, seq=S, tn=tn)
    attn = _flash_attention(qkv, batch=B, seq=S, head_dim=d)
    out = _o_proj(attn, wo, seq=S)
    return {"hidden_states": out.reshape(S, B, Hd),
            "sequence_mask": sequence_mask}


# speedup vs baseline: 3.8659x; 1.3613x over previous
import functools

import jax
import jax.numpy as jnp
import numpy as np
from jax.experimental import pallas as pl
from jax.experimental.pallas import tpu as pltpu

NUM_HEADS = 16
NUM_KV_HEADS = 4
GROUP = NUM_HEADS // NUM_KV_HEADS


def _rotary_epilogue_tables_np(seq_len, dim, n_heads_tile, scale,
                               theta=10000.0):
    half = dim // 2
    inv_freq = 1.0 / (theta ** (np.arange(0, dim, 2, dtype=np.float64) / dim))
    ang = np.outer(np.arange(seq_len, dtype=np.float64), inv_freq)
    cos = np.cos(ang)
    sin = np.sin(ang)
    cos_cat = np.concatenate([cos, cos], axis=-1)
    sin_cat = np.concatenate([-sin, sin], axis=-1)
    sin_lo = np.concatenate([-sin, np.zeros_like(sin)], axis=-1)
    sin_hi = np.concatenate([np.zeros_like(sin), sin], axis=-1)
    del cos_cat, half

    def tile_q(t):
        return np.tile(t, (1, n_heads_tile)) * scale

    nk = n_heads_tile // 2
    ones_v = np.ones((seq_len, nk * dim))
    zeros_v = np.zeros((seq_len, nk * dim))

    cos_tab = np.concatenate(
        [tile_q(np.concatenate([cos, cos], -1)),
         np.concatenate([np.tile(np.concatenate([cos, cos], -1), (1, nk)),
                         ones_v], -1)], axis=-1)
    sin_lo_tab = np.concatenate(
        [tile_q(sin_lo),
         np.concatenate([np.tile(sin_lo, (1, nk)), zeros_v], -1)], axis=-1)
    sin_hi_tab = np.concatenate(
        [tile_q(sin_hi),
         np.concatenate([np.tile(sin_hi, (1, nk)), zeros_v], -1)], axis=-1)
    del sin_cat
    return (cos_tab.astype(np.float32), sin_lo_tab.astype(np.float32),
            sin_hi_tab.astype(np.float32))


def _qkv_rot_kernel(x_ref, w_ref, cos_ref, slo_ref, shi_ref, o_ref, xb_scr,
                    *, half, tn):
    j = pl.program_id(1)

    @pl.when(j == 0)
    def _cast():
        xb_scr[...] = x_ref[...].astype(xb_scr.dtype)

    acc = jnp.dot(xb_scr[...], w_ref[:, pl.ds(j * tn, tn)],
                  preferred_element_type=jnp.float32)
    sel = (j // 2) * tn
    cos = cos_ref[:, pl.ds(sel, tn)]
    slo = slo_ref[:, pl.ds(sel, tn)]
    shi = shi_ref[:, pl.ds(sel, tn)]
    r_lo = pltpu.roll(acc, shift=tn - half, axis=1)
    r_hi = pltpu.roll(acc, shift=half, axis=1)
    out = acc * cos + r_lo * slo + r_hi * shi
    o_ref[...] = out.astype(o_ref.dtype)


def _qkv_proj_rotary(x2d, w, tabs, *, seq, tn=1024):
    S, BHd = x2d.shape
    K, N = w.shape
    B = BHd // K
    tm = seq
    d = 2 * tn
    kern = functools.partial(_qkv_rot_kernel, half=64, tn=tn)
    return pl.pallas_call(
        kern,
        out_shape=jax.ShapeDtypeStruct((B * seq, N), jnp.bfloat16),
        grid=(B, N // tn),
        in_specs=[
            pl.BlockSpec((S, K), lambda i, j: (0, i)),
            pl.BlockSpec((K, N), lambda i, j: (0, 0)),
            pl.BlockSpec((seq, d), lambda i, j: (0, 0)),
            pl.BlockSpec((seq, d), lambda i, j: (0, 0)),
            pl.BlockSpec((seq, d), lambda i, j: (0, 0)),
        ],
        out_specs=pl.BlockSpec((tm, tn), lambda i, j: (i, j)),
        scratch_shapes=[pltpu.VMEM((S, K), jnp.bfloat16)],
        compiler_params=pltpu.CompilerParams(
            dimension_semantics=("parallel", "arbitrary"),
            vmem_limit_bytes=56 * 1024 * 1024),
    )(x2d, w, *tabs)


def _attn_kernel(q_ref, k_ref, v_ref, o_ref, *, tq, d, n_q):
    qi = pl.program_id(2)

    def _body(nb):
        L = nb * tq
        k = k_ref[0:L, :]
        v = v_ref[0:L, :]
        row = jax.lax.broadcasted_iota(jnp.int32, (tq, L), 0)
        col = jax.lax.broadcasted_iota(jnp.int32, (tq, L), 1)
        keep = col <= row + (nb - 1) * tq
        for u in range(GROUP):
            sl = slice(u * d, (u + 1) * d)
            s = jax.lax.dot_general(q_ref[:, sl], k, (((1,), (1,)), ((), ())),
                                    preferred_element_type=jnp.float32)
            s = jnp.where(keep, s, -jnp.inf)
            m = jnp.max(s, axis=-1, keepdims=True)
            p = jnp.exp(s - m)
            inv = pl.reciprocal(jnp.sum(p, axis=-1, keepdims=True), approx=True)
            pv = jnp.dot(p.astype(v.dtype), v,
                         preferred_element_type=jnp.float32)
            o_ref[:, sl] = (pv * inv).astype(o_ref.dtype)

    for nb in range(1, n_q + 1):
        pl.when(qi == nb - 1)(functools.partial(_body, nb))


def _flash_attention(qkv, *, batch, seq, head_dim, tq=512):
    d = head_dim
    n_q = seq // tq

    return pl.pallas_call(
        functools.partial(_attn_kernel, tq=tq, d=d, n_q=n_q),
        out_shape=jax.ShapeDtypeStruct((batch * seq, NUM_HEADS * d),
                                       jnp.bfloat16),
        grid=(batch, NUM_KV_HEADS, n_q),
        in_specs=[
            pl.BlockSpec((tq, GROUP * d),
                         lambda b, g, qi, _n=n_q: (b * _n + qi, g)),
            pl.BlockSpec((seq, d), lambda b, g, qi: (b, NUM_HEADS + g)),
            pl.BlockSpec((seq, d),
                         lambda b, g, qi: (b, NUM_HEADS + NUM_KV_HEADS + g)),
        ],
        out_specs=pl.BlockSpec((tq, GROUP * d),
                               lambda b, g, qi, _n=n_q: (b * _n + qi, g)),
        compiler_params=pltpu.CompilerParams(
            dimension_semantics=("parallel", "parallel", "arbitrary"),
            vmem_limit_bytes=56 * 1024 * 1024),
    )(qkv, qkv, qkv)


def _mm_kernel(x_ref, w_ref, o_ref, *, tn):
    j = pl.program_id(1)
    o_ref[...] = jnp.dot(x_ref[...], w_ref[:, pl.ds(j * tn, tn)],
                         preferred_element_type=jnp.float32).astype(o_ref.dtype)


def _o_proj(x, w, *, seq, tn=1024, out_dtype=jnp.float32):
    M, K = x.shape
    _, N = w.shape
    B = M // seq
    nj = N // tn
    return pl.pallas_call(
        functools.partial(_mm_kernel, tn=tn),
        out_shape=jax.ShapeDtypeStruct((seq, B * N), out_dtype),
        grid=(B, nj),
        in_specs=[
            pl.BlockSpec((seq, K), lambda i, j: (i, 0)),
            pl.BlockSpec((K, N), lambda i, j: (0, 0)),
        ],
        out_specs=pl.BlockSpec((seq, tn), lambda i, j, _nj=nj: (0, i * _nj + j)),
        compiler_params=pltpu.CompilerParams(
            dimension_semantics=("parallel", "arbitrary"),
            vmem_limit_bytes=56 * 1024 * 1024),
    )(x, w)


def kernel(w_qkv, w_o, hidden_states, sequence_mask):
    S, B, Hd = hidden_states.shape
    d = Hd // NUM_HEADS
    scale = 1.0 / (d ** 0.5)

    x2d = hidden_states.reshape(S, B * Hd)
    wq = w_qkv.astype(jnp.bfloat16)
    wo = w_o.astype(jnp.bfloat16)

    tn = 1024
    tabs_np = _rotary_epilogue_tables_np(S, d, tn // d, scale)
    tabs = tuple(jnp.asarray(t, jnp.bfloat16) for t in tabs_np)

    qkv = _qkv_proj_rotary(x2d, wq, tabs, seq=S, tn=tn)
    attn = _flash_attention(qkv, batch=B, seq=S, head_dim=d)
    out = _o_proj(attn, wo, seq=S)
    return {"hidden_states": out.reshape(S, B, Hd),
            "sequence_mask": sequence_mask}


# speedup vs baseline: 4.0627x; 1.0509x over previous
import functools

import jax
import jax.numpy as jnp
import numpy as np
from jax.experimental import pallas as pl
from jax.experimental.pallas import tpu as pltpu

NUM_HEADS = 16
NUM_KV_HEADS = 4
GROUP = NUM_HEADS // NUM_KV_HEADS


def _group_tables_np(seq_len, dim, scale, theta=10000.0):
    inv_freq = 1.0 / (theta ** (np.arange(0, dim, 2, dtype=np.float64) / dim))
    ang = np.outer(np.arange(seq_len, dtype=np.float64), inv_freq)
    cos = np.cos(ang)
    sin = np.sin(ang)
    cos_cat = np.concatenate([cos, cos], axis=-1)
    sin_lo = np.concatenate([-sin, np.zeros_like(sin)], axis=-1)
    sin_hi = np.concatenate([np.zeros_like(sin), sin], axis=-1)
    ones = np.ones((seq_len, dim))
    zeros = np.zeros((seq_len, dim))

    def build(q_pat, k_pat, v_pat):
        return np.concatenate([np.tile(q_pat, (1, GROUP)) * scale,
                               k_pat, v_pat], axis=-1).astype(np.float32)

    return (build(cos_cat, cos_cat, ones),
            build(sin_lo, sin_lo, zeros),
            build(sin_hi, sin_hi, zeros))


def _fused_kernel(x_ref, wq_ref, wo_ref, cos_ref, slo_ref, shi_ref, o_ref,
                  qkv_scr, attn_scr, *, seq, d, tq, gw):
    g = pl.program_id(1)
    half = d // 2
    n_q = seq // tq

    for qi in range(n_q):
        rows = slice(qi * tq, (qi + 1) * tq)
        acc = jnp.dot(x_ref[rows, :], wq_ref[...],
                      preferred_element_type=jnp.float32)
        r_lo = pltpu.roll(acc, shift=gw - half, axis=1)
        r_hi = pltpu.roll(acc, shift=half, axis=1)
        qkv_scr[rows, :] = (acc * cos_ref[rows, :] + r_lo * slo_ref[rows, :]
                            + r_hi * shi_ref[rows, :]).astype(qkv_scr.dtype)

    for qi in range(n_q):
        L = (qi + 1) * tq
        k = qkv_scr[0:L, GROUP * d:GROUP * d + d]
        v = qkv_scr[0:L, GROUP * d + d:GROUP * d + 2 * d]
        row = jax.lax.broadcasted_iota(jnp.int32, (tq, L), 0)
        col = jax.lax.broadcasted_iota(jnp.int32, (tq, L), 1)
        keep = col <= row + qi * tq
        for u in range(GROUP):
            q_u = qkv_scr[qi * tq:(qi + 1) * tq, u * d:(u + 1) * d]
            s = jax.lax.dot_general(q_u, k, (((1,), (1,)), ((), ())),
                                    preferred_element_type=jnp.float32)
            s = jnp.where(keep, s, -jnp.inf)
            m = jnp.max(s, axis=-1, keepdims=True)
            p = jnp.exp(s - m)
            inv = pl.reciprocal(jnp.sum(p, axis=-1, keepdims=True),
                                approx=True)
            pv = jnp.dot(p.astype(jnp.bfloat16), v,
                         preferred_element_type=jnp.float32)
            attn_scr[qi * tq:(qi + 1) * tq,
                     pl.ds(g * GROUP * d + u * d, d)] = (
                         pv * inv).astype(attn_scr.dtype)

    @pl.when(g == pl.num_programs(1) - 1)
    def _o_proj():
        o_ref[...] = jnp.dot(attn_scr[...], wo_ref[...],
                             preferred_element_type=jnp.float32
                             ).astype(o_ref.dtype)


def kernel(w_qkv, w_o, hidden_states, sequence_mask):
    S, B, Hd = hidden_states.shape
    d = Hd // NUM_HEADS
    scale = 1.0 / (d ** 0.5)
    gw = (GROUP + 2) * d

    x2d = hidden_states.reshape(S, B * Hd).astype(jnp.bfloat16)

    qc, kc = NUM_HEADS * d, NUM_KV_HEADS * d
    wq_re = jnp.concatenate(
        [jnp.concatenate(
            [w_qkv[:, g * GROUP * d:(g + 1) * GROUP * d],
             w_qkv[:, qc + g * d:qc + (g + 1) * d],
             w_qkv[:, qc + kc + g * d:qc + kc + (g + 1) * d]], axis=1)
         for g in range(NUM_KV_HEADS)], axis=1).astype(jnp.bfloat16)
    wo_bf = w_o.astype(jnp.bfloat16)

    tabs_np = _group_tables_np(S, d, scale)
    tabs = tuple(jnp.asarray(t, jnp.bfloat16) for t in tabs_np)

    kern = functools.partial(_fused_kernel, seq=S, d=d, tq=512, gw=gw)
    out = pl.pallas_call(
        kern,
        out_shape=jax.ShapeDtypeStruct((S, B * Hd), jnp.float32),
        grid=(B, NUM_KV_HEADS),
        in_specs=[
            pl.BlockSpec((S, Hd), lambda i, g: (0, i)),
            pl.BlockSpec((Hd, gw), lambda i, g: (0, g)),
            pl.BlockSpec(wo_bf.shape, lambda i, g: (0, 0)),
            pl.BlockSpec((S, gw), lambda i, g: (0, 0)),
            pl.BlockSpec((S, gw), lambda i, g: (0, 0)),
            pl.BlockSpec((S, gw), lambda i, g: (0, 0)),
        ],
        out_specs=pl.BlockSpec((S, Hd), lambda i, g: (0, i)),
        scratch_shapes=[
            pltpu.VMEM((S, gw), jnp.bfloat16),
            pltpu.VMEM((S, NUM_HEADS * d), jnp.bfloat16),
        ],
        compiler_params=pltpu.CompilerParams(
            dimension_semantics=("parallel", "arbitrary"),
            vmem_limit_bytes=67043328),
    )(x2d, wq_re, wo_bf, *tabs)

    return {"hidden_states": out.reshape(S, B, Hd),
            "sequence_mask": sequence_mask}
